# baseline (speedup 1.0000x reference)
import functools

import jax
import jax.numpy as jnp
from jax import lax
from jax.experimental import pallas as pl
from jax.experimental.pallas import tpu as pltpu
from jax.experimental.pallas import tpu_sc as plsc

B = 16384
D_ITEM = 64
D_SPARSE = 32
MM_DIM = 128
D_MM = 32
DNN_HID = 256
HID_OUT = 128

_NC = 2
_NS = 16
_NW = _NC * _NS
_BPW = B // _NW
_CHUNK = 128
_NCHUNK = _BPW // _CHUNK

_BLK = 1024
_GRID = B // _BLK

_DIMS = (D_ITEM, D_SPARSE, D_SPARSE, D_SPARSE, D_SPARSE)


def _sc_gather_body(seq_ref, cate_ref, brand_ref, shop_ref, tag_ref,
                    t_item, t_cate, t_brand, t_shop, t_tag,
                    o_item, o_cate, o_brand, o_shop, o_tag,
                    idx_v, rows_v, sem):
    wid = lax.axis_index("s") * _NC + lax.axis_index("c")
    base = wid * _BPW
    in_refs = (seq_ref, cate_ref, brand_ref, shop_ref, tag_ref)
    tabs = (t_item, t_cate, t_brand, t_shop, t_tag)
    outs = (o_item, o_cate, o_brand, o_shop, o_tag)

    for t, d_t in enumerate(_DIMS):
        pltpu.sync_copy(in_refs[t].at[pl.ds(base, _BPW)], idx_v)

        def dma_body(c, carry, *, t=t, d_t=d_t):
            idx_chunk = idx_v.at[pl.ds(c * _CHUNK, _CHUNK)]
            for d in range(d_t):
                pltpu.async_copy(
                    tabs[t].at[d].at[idx_chunk],
                    rows_v.at[d, pl.ds(c * _CHUNK, _CHUNK)],
                    sem)
            return carry

        lax.fori_loop(0, _NCHUNK, dma_body, 0, unroll=False)
        pltpu.make_async_copy(
            tabs[t].at[pl.ds(0, d_t), pl.ds(0, _BPW)],
            rows_v.at[pl.ds(0, d_t)],
            sem,
        ).wait()
        pltpu.sync_copy(rows_v.at[pl.ds(0, d_t)],
                        outs[t].at[:, pl.ds(base, _BPW)])


@jax.jit
def _sc_gather(seq_id, cate_id, brand_id, shop_id, tag_id,
               t_item, t_cate, t_brand, t_shop, t_tag):
    mesh = plsc.VectorSubcoreMesh(core_axis_name="c", subcore_axis_name="s")
    f32 = jnp.float32
    out_type = [
        jax.ShapeDtypeStruct((D_ITEM, B), f32),
        jax.ShapeDtypeStruct((D_SPARSE, B), f32),
        jax.ShapeDtypeStruct((D_SPARSE, B), f32),
        jax.ShapeDtypeStruct((D_SPARSE, B), f32),
        jax.ShapeDtypeStruct((D_SPARSE, B), f32),
    ]
    scratch = [
        pltpu.VMEM((_BPW,), jnp.int32),
        pltpu.VMEM((D_ITEM, _BPW), f32),
        pltpu.SemaphoreType.DMA,
    ]
    return pl.kernel(
        _sc_gather_body,
        out_type=out_type,
        mesh=mesh,
        scratch_types=scratch,
        compiler_params=pltpu.CompilerParams(use_tc_tiling_on_sc=False),
    )(seq_id, cate_id, brand_id, shop_id, tag_id,
      t_item, t_cate, t_brand, t_shop, t_tag)


def _dot0(a, b):
    return lax.dot_general(a, b, (((0,), (0,)), ((), ())),
                           preferred_element_type=jnp.float32)


def _mlp_body(gi, gc, gb, gs, gt, dns, mm,
              mmW, mmb, w1i, w1c, w1b, w1s, w1t, w1d, w1m, b1, w2, b2,
              out):
    f32 = jnp.float32
    acc = _dot0(gi[...], w1i[...])
    acc += _dot0(gc[...], w1c[...])
    acc += _dot0(gb[...], w1b[...])
    acc += _dot0(gs[...], w1s[...])
    acc += _dot0(gt[...], w1t[...])
    acc += jnp.dot(dns[...], w1d[...], preferred_element_type=f32)
    mmp = jnp.dot(mm[...], mmW[...], preferred_element_type=f32) + mmb[...]
    acc += jnp.dot(mmp, w1m[...], preferred_element_type=f32)
    acc += b1[...]
    h = jnp.maximum(acc, 0.0)
    out[...] = jnp.dot(h, w2[...], preferred_element_type=f32) + b2[...]


def _full(shape):
    return pl.BlockSpec(shape, lambda i: (0, 0))


def _mlp(gi, gc, gb, gs, gt, dns, mm, mmW, mmb,
         w1i, w1c, w1b, w1s, w1t, w1d, w1m, b1, w2, b2):
    blk_t = lambda d: pl.BlockSpec((d, _BLK), lambda i: (0, i))
    blk = lambda d: pl.BlockSpec((_BLK, d), lambda i: (i, 0))
    in_specs = [
        blk_t(D_ITEM), blk_t(D_SPARSE), blk_t(D_SPARSE), blk_t(D_SPARSE),
        blk_t(D_SPARSE),
        blk(3), blk(MM_DIM),
        _full((MM_DIM, D_MM)), _full((1, D_MM)),
        _full((D_ITEM, DNN_HID)),
        _full((D_SPARSE, DNN_HID)), _full((D_SPARSE, DNN_HID)),
        _full((D_SPARSE, DNN_HID)), _full((D_SPARSE, DNN_HID)),
        _full((3, DNN_HID)), _full((D_MM, DNN_HID)),
        _full((1, DNN_HID)),
        _full((DNN_HID, HID_OUT)), _full((1, HID_OUT)),
    ]
    return pl.pallas_call(
        _mlp_body,
        grid=(_GRID,),
        in_specs=in_specs,
        out_specs=pl.BlockSpec((_BLK, HID_OUT), lambda i: (i, 0)),
        out_shape=jax.ShapeDtypeStruct((B, HID_OUT), jnp.float32),
        compiler_params=pltpu.CompilerParams(
            dimension_semantics=("arbitrary",)),
    )(gi, gc, gb, gs, gt, dns, mm, mmW, mmb,
      w1i, w1c, w1b, w1s, w1t, w1d, w1m, b1, w2, b2)


def kernel(seq_id, cate_id, brand_id, shop_id, tag_id,
           dense_0, dense_1, dense_2, mm_emb_0,
           emb_item, emb_cate, emb_brand, emb_shop, emb_tag,
           mm_W, mm_b, W1, b1, W2, b2):
    i32 = jnp.int32
    giT, gcT, gbT, gsT, gtT = _sc_gather(
        seq_id.astype(i32), cate_id.astype(i32), brand_id.astype(i32),
        shop_id.astype(i32), tag_id.astype(i32),
        emb_item.T, emb_cate.T, emb_brand.T, emb_shop.T, emb_tag.T)

    dns = jnp.stack([dense_0, dense_1, dense_2], axis=1)
    w1i = W1[:D_ITEM]
    o = D_ITEM
    w1c = W1[o:o + D_SPARSE]; o += D_SPARSE
    w1b = W1[o:o + D_SPARSE]; o += D_SPARSE
    w1s = W1[o:o + D_SPARSE]; o += D_SPARSE
    w1t = W1[o:o + D_SPARSE]; o += D_SPARSE
    w1d = W1[o:o + 3]; o += 3
    w1m = W1[o:o + D_MM]

    return _mlp(giT, gcT, gbT, gsT, gtT, dns, mm_emb_0,
                mm_W, mm_b.reshape(1, -1),
                w1i, w1c, w1b, w1s, w1t, w1d, w1m,
                b1.reshape(1, -1), W2, b2.reshape(1, -1))


# speedup vs baseline: 6.5721x; 6.5721x over previous
import functools

import jax
import jax.numpy as jnp
from jax import lax
from jax.experimental import pallas as pl
from jax.experimental.pallas import tpu as pltpu
from jax.experimental.pallas import tpu_sc as plsc

B = 16384
D_ITEM = 64
D_SPARSE = 32
D_SP4 = 4 * D_SPARSE
MM_DIM = 128
D_MM = 32
DNN_HID = 256
HID_OUT = 128

_NC = 2
_NS = 16
_NW = _NC * _NS
_BPW = B // _NW
_CHUNK = 128
_NCHUNK = _BPW // _CHUNK

_BLK = 1024
_GRID = B // _BLK


def _sc_gather_body(seq_ref, cate_ref, brand_ref, shop_ref, tag_ref,
                    t_item, t_sp4,
                    o_item, o_cate, o_brand, o_shop, o_tag,
                    idx0, idx1, idx2, idx3, idx4,
                    r_item, r_sp,
                    s_item, s_sp):
    wid = lax.axis_index("s") * _NC + lax.axis_index("c")
    base = wid * _BPW
    idx_refs = (idx0, idx1, idx2, idx3, idx4)
    in_refs = (seq_ref, cate_ref, brand_ref, shop_ref, tag_ref)
    outs = (o_item, o_cate, o_brand, o_shop, o_tag)

    for i in range(5):
        pltpu.sync_copy(in_refs[i].at[pl.ds(base, _BPW)], idx_refs[i])

    item_handles = []
    for j in range(_NCHUNK):
        item_handles.append(pltpu.async_copy(
            t_item.at[idx_refs[0].at[pl.ds(j * _CHUNK, _CHUNK)]],
            r_item.at[pl.ds(j * _CHUNK, _CHUNK)],
            s_item))

    for i in range(1, 5):
        hs = []
        for j in range(_NCHUNK):
            hs.append(pltpu.async_copy(
                t_sp4.at[idx_refs[i].at[pl.ds(j * _CHUNK, _CHUNK)]],
                r_sp.at[pl.ds(j * _CHUNK, _CHUNK)],
                s_sp))
        for h in hs:
            h.wait()
        col = (i - 1) * D_SPARSE
        pltpu.sync_copy(r_sp.at[:, pl.ds(col, D_SPARSE)],
                        outs[i].at[pl.ds(base, _BPW)])

    for h in item_handles:
        h.wait()
    pltpu.sync_copy(r_item, o_item.at[pl.ds(base, _BPW)])


@jax.jit
def _sc_gather(seq_id, cate_id, brand_id, shop_id, tag_id, t_item, t_sp4):
    mesh = plsc.VectorSubcoreMesh(core_axis_name="c", subcore_axis_name="s")
    f32 = jnp.float32
    out_type = [
        jax.ShapeDtypeStruct((B, D_ITEM), f32),
        jax.ShapeDtypeStruct((B, D_SPARSE), f32),
        jax.ShapeDtypeStruct((B, D_SPARSE), f32),
        jax.ShapeDtypeStruct((B, D_SPARSE), f32),
        jax.ShapeDtypeStruct((B, D_SPARSE), f32),
    ]
    scratch = (
        [pltpu.VMEM((_BPW,), jnp.int32) for _ in range(5)]
        + [pltpu.VMEM((_BPW, D_ITEM), f32), pltpu.VMEM((_BPW, D_SP4), f32)]
        + [pltpu.SemaphoreType.DMA, pltpu.SemaphoreType.DMA]
    )
    return pl.kernel(
        _sc_gather_body,
        out_type=out_type,
        mesh=mesh,
        scratch_types=scratch,
        compiler_params=pltpu.CompilerParams(use_tc_tiling_on_sc=False),
    )(seq_id, cate_id, brand_id, shop_id, tag_id, t_item, t_sp4)


def _mlp_body(gi, gc, gb, gs, gt, dns, mm,
              mmW, mmb, w1i, w1c, w1b, w1s, w1t, w1d, w1m, b1, w2, b2,
              out):
    f32 = jnp.float32
    acc = jnp.dot(gi[...], w1i[...], preferred_element_type=f32)
    acc += jnp.dot(gc[...], w1c[...], preferred_element_type=f32)
    acc += jnp.dot(gb[...], w1b[...], preferred_element_type=f32)
    acc += jnp.dot(gs[...], w1s[...], preferred_element_type=f32)
    acc += jnp.dot(gt[...], w1t[...], preferred_element_type=f32)
    acc += jnp.dot(dns[...], w1d[...], preferred_element_type=f32)
    mmp = jnp.dot(mm[...], mmW[...], preferred_element_type=f32) + mmb[...]
    acc += jnp.dot(mmp, w1m[...], preferred_element_type=f32)
    acc += b1[...]
    h = jnp.maximum(acc, 0.0)
    out[...] = jnp.dot(h, w2[...], preferred_element_type=f32) + b2[...]


def _full(shape):
    return pl.BlockSpec(shape, lambda i: (0, 0))


def _mlp(gi, gc, gb, gs, gt, dns, mm, mmW, mmb,
         w1i, w1c, w1b, w1s, w1t, w1d, w1m, b1, w2, b2):
    blk = lambda d: pl.BlockSpec((_BLK, d), lambda i: (i, 0))
    in_specs = [
        blk(D_ITEM), blk(D_SPARSE), blk(D_SPARSE), blk(D_SPARSE), blk(D_SPARSE),
        blk(3), blk(MM_DIM),
        _full((MM_DIM, D_MM)), _full((1, D_MM)),
        _full((D_ITEM, DNN_HID)),
        _full((D_SPARSE, DNN_HID)), _full((D_SPARSE, DNN_HID)),
        _full((D_SPARSE, DNN_HID)), _full((D_SPARSE, DNN_HID)),
        _full((3, DNN_HID)), _full((D_MM, DNN_HID)),
        _full((1, DNN_HID)),
        _full((DNN_HID, HID_OUT)), _full((1, HID_OUT)),
    ]
    return pl.pallas_call(
        _mlp_body,
        grid=(_GRID,),
        in_specs=in_specs,
        out_specs=pl.BlockSpec((_BLK, HID_OUT), lambda i: (i, 0)),
        out_shape=jax.ShapeDtypeStruct((B, HID_OUT), jnp.float32),
        compiler_params=pltpu.CompilerParams(
            dimension_semantics=("arbitrary",)),
    )(gi, gc, gb, gs, gt, dns, mm, mmW, mmb,
      w1i, w1c, w1b, w1s, w1t, w1d, w1m, b1, w2, b2)


def kernel(seq_id, cate_id, brand_id, shop_id, tag_id,
           dense_0, dense_1, dense_2, mm_emb_0,
           emb_item, emb_cate, emb_brand, emb_shop, emb_tag,
           mm_W, mm_b, W1, b1, W2, b2):
    i32 = jnp.int32
    t_sp4 = jnp.concatenate([emb_cate, emb_brand, emb_shop, emb_tag], axis=1)
    gi, gc, gb, gs, gt = _sc_gather(
        seq_id.astype(i32), cate_id.astype(i32), brand_id.astype(i32),
        shop_id.astype(i32), tag_id.astype(i32),
        emb_item, t_sp4)

    dns = jnp.stack([dense_0, dense_1, dense_2], axis=1)
    w1i = W1[:D_ITEM]
    o = D_ITEM
    w1c = W1[o:o + D_SPARSE]; o += D_SPARSE
    w1b = W1[o:o + D_SPARSE]; o += D_SPARSE
    w1s = W1[o:o + D_SPARSE]; o += D_SPARSE
    w1t = W1[o:o + D_SPARSE]; o += D_SPARSE
    w1d = W1[o:o + 3]; o += 3
    w1m = W1[o:o + D_MM]

    return _mlp(gi, gc, gb, gs, gt, dns, mm_emb_0,
                mm_W, mm_b.reshape(1, -1),
                w1i, w1c, w1b, w1s, w1t, w1d, w1m,
                b1.reshape(1, -1), W2, b2.reshape(1, -1))


# speedup vs baseline: 6.8226x; 1.0381x over previous
import functools

import jax
import jax.numpy as jnp
from jax import lax
from jax.experimental import pallas as pl
from jax.experimental.pallas import tpu as pltpu
from jax.experimental.pallas import tpu_sc as plsc

B = 16384
D_ITEM = 64
D_SPARSE = 32
MM_DIM = 128
D_MM = 32
DNN_HID = 256
HID_OUT = 128

_NC = 2
_NS = 16
_NW = _NC * _NS
_BPW = B // _NW
_CHUNK = 128
_NCHUNK = _BPW // _CHUNK

_BLK = 1024
_GRID = B // _BLK


def _sc_gather_body(seq_ref, cate_ref, brand_ref, shop_ref, tag_ref,
                    t_item, t_cate, t_brand, t_shop, t_tag,
                    o_item, o_cate, o_brand, o_shop, o_tag,
                    idx0, idx1, idx2, idx3, idx4,
                    r0, r1, r2, r3, r4,
                    s0, s1, s2, s3, s4):
    wid = lax.axis_index("s") * _NC + lax.axis_index("c")
    base = wid * _BPW
    idx_refs = (idx0, idx1, idx2, idx3, idx4)
    in_refs = (seq_ref, cate_ref, brand_ref, shop_ref, tag_ref)
    tabs = (t_item, t_cate, t_brand, t_shop, t_tag)
    rows = (r0, r1, r2, r3, r4)
    outs = (o_item, o_cate, o_brand, o_shop, o_tag)
    sems = (s0, s1, s2, s3, s4)

    for i in range(5):
        pltpu.sync_copy(in_refs[i].at[pl.ds(base, _BPW)], idx_refs[i])

    handles = []
    for i in range(5):
        per_tab = []
        for j in range(_NCHUNK):
            h = pltpu.async_copy(
                tabs[i].at[idx_refs[i].at[pl.ds(j * _CHUNK, _CHUNK)]],
                rows[i].at[pl.ds(j * _CHUNK, _CHUNK)],
                sems[i])
            per_tab.append(h)
        handles.append(per_tab)

    for i in range(5):
        for h in handles[i]:
            h.wait()
        pltpu.sync_copy(rows[i], outs[i].at[pl.ds(base, _BPW)])


@jax.jit
def _sc_gather(seq_id, cate_id, brand_id, shop_id, tag_id,
               emb_item, emb_cate, emb_brand, emb_shop, emb_tag):
    mesh = plsc.VectorSubcoreMesh(core_axis_name="c", subcore_axis_name="s")
    f32 = jnp.float32
    out_type = [
        jax.ShapeDtypeStruct((B, D_ITEM), f32),
        jax.ShapeDtypeStruct((B, D_SPARSE), f32),
        jax.ShapeDtypeStruct((B, D_SPARSE), f32),
        jax.ShapeDtypeStruct((B, D_SPARSE), f32),
        jax.ShapeDtypeStruct((B, D_SPARSE), f32),
    ]
    scratch = (
        [pltpu.VMEM((_BPW,), jnp.int32) for _ in range(5)]
        + [pltpu.VMEM((_BPW, D_ITEM), f32)]
        + [pltpu.VMEM((_BPW, D_SPARSE), f32) for _ in range(4)]
        + [pltpu.SemaphoreType.DMA for _ in range(5)]
    )
    return pl.kernel(
        _sc_gather_body,
        out_type=out_type,
        mesh=mesh,
        scratch_types=scratch,
        compiler_params=pltpu.CompilerParams(use_tc_tiling_on_sc=False),
    )(seq_id, cate_id, brand_id, shop_id, tag_id,
      emb_item, emb_cate, emb_brand, emb_shop, emb_tag)


def _mlp_body(gi, gc, gb, gs, gt, dns, mm,
              mmW, mmb, w1i, w1c, w1b, w1s, w1t, w1d, w1m, b1, w2, b2,
              out):
    f32 = jnp.float32
    acc = jnp.dot(gi[...], w1i[...], preferred_element_type=f32)
    acc += jnp.dot(gc[...], w1c[...], preferred_element_type=f32)
    acc += jnp.dot(gb[...], w1b[...], preferred_element_type=f32)
    acc += jnp.dot(gs[...], w1s[...], preferred_element_type=f32)
    acc += jnp.dot(gt[...], w1t[...], preferred_element_type=f32)
    acc += jnp.dot(dns[...], w1d[...], preferred_element_type=f32)
    mmp = jnp.dot(mm[...], mmW[...], preferred_element_type=f32) + mmb[...]
    acc += jnp.dot(mmp, w1m[...], preferred_element_type=f32)
    acc += b1[...]
    h = jnp.maximum(acc, 0.0)
    out[...] = jnp.dot(h, w2[...], preferred_element_type=f32) + b2[...]


def _full(shape):
    return pl.BlockSpec(shape, lambda i: (0, 0))


def _mlp(gi, gc, gb, gs, gt, dns, mm, mmW, mmb,
         w1i, w1c, w1b, w1s, w1t, w1d, w1m, b1, w2, b2):
    blk = lambda d: pl.BlockSpec((_BLK, d), lambda i: (i, 0))
    in_specs = [
        blk(D_ITEM), blk(D_SPARSE), blk(D_SPARSE), blk(D_SPARSE), blk(D_SPARSE),
        blk(3), blk(MM_DIM),
        _full((MM_DIM, D_MM)), _full((1, D_MM)),
        _full((D_ITEM, DNN_HID)),
        _full((D_SPARSE, DNN_HID)), _full((D_SPARSE, DNN_HID)),
        _full((D_SPARSE, DNN_HID)), _full((D_SPARSE, DNN_HID)),
        _full((3, DNN_HID)), _full((D_MM, DNN_HID)),
        _full((1, DNN_HID)),
        _full((DNN_HID, HID_OUT)), _full((1, HID_OUT)),
    ]
    return pl.pallas_call(
        _mlp_body,
        grid=(_GRID,),
        in_specs=in_specs,
        out_specs=pl.BlockSpec((_BLK, HID_OUT), lambda i: (i, 0)),
        out_shape=jax.ShapeDtypeStruct((B, HID_OUT), jnp.float32),
        compiler_params=pltpu.CompilerParams(
            dimension_semantics=("arbitrary",)),
    )(gi, gc, gb, gs, gt, dns, mm, mmW, mmb,
      w1i, w1c, w1b, w1s, w1t, w1d, w1m, b1, w2, b2)


def kernel(seq_id, cate_id, brand_id, shop_id, tag_id,
           dense_0, dense_1, dense_2, mm_emb_0,
           emb_item, emb_cate, emb_brand, emb_shop, emb_tag,
           mm_W, mm_b, W1, b1, W2, b2):
    i32 = jnp.int32
    gi, gc, gb, gs, gt = _sc_gather(
        seq_id.astype(i32), cate_id.astype(i32), brand_id.astype(i32),
        shop_id.astype(i32), tag_id.astype(i32),
        emb_item, emb_cate, emb_brand, emb_shop, emb_tag)

    dns = jnp.stack([dense_0, dense_1, dense_2], axis=1)
    w1i = W1[:D_ITEM]
    o = D_ITEM
    w1c = W1[o:o + D_SPARSE]; o += D_SPARSE
    w1b = W1[o:o + D_SPARSE]; o += D_SPARSE
    w1s = W1[o:o + D_SPARSE]; o += D_SPARSE
    w1t = W1[o:o + D_SPARSE]; o += D_SPARSE
    w1d = W1[o:o + 3]; o += 3
    w1m = W1[o:o + D_MM]

    return _mlp(gi, gc, gb, gs, gt, dns, mm_emb_0,
                mm_W, mm_b.reshape(1, -1),
                w1i, w1c, w1b, w1s, w1t, w1d, w1m,
                b1.reshape(1, -1), W2, b2.reshape(1, -1))
